# baseline (device time: 47396 ns/iter reference)
import jax
import jax.numpy as jnp
from jax import lax
from jax.experimental import pallas as pl
from jax.experimental.pallas import tpu as pltpu

NCHUNK = 4


def kernel(O, Wo):
    B, S, H, D = O.shape
    K = H * D
    N = Wo.shape[1]
    S_out = S // 2
    Q = S_out // 2
    SPLIT = NCHUNK // B
    R = Q // SPLIT

    OT = O.transpose(0, 2, 3, 1).reshape(B, K, S)

    def body(o_ref, wo_ref, out_ref, wo_bf, o_vm, x_send, x_recv, y_send,
             y_recv, acc, qb, in_dma_sems, x_send_sems, x_recv_sems,
             y_send_sems, y_recv_sems, out_dma_sems):
        my_x = lax.axis_index("x")
        my_y = lax.axis_index("y")
        peer_x = 1 - my_x
        peer_y = 1 - my_y

        def chunk(c):
            return c // SPLIT, (c % SPLIT) * R

        in_dmas = []
        for c in range(NCHUNK):
            b, r = chunk(c)
            dma = pltpu.make_async_copy(
                o_ref.at[b, :, pl.ds(peer_x * S_out + my_y * Q + r, R)],
                o_vm.at[c],
                in_dma_sems.at[c],
            )
            dma.start()
            in_dmas.append(dma)
        for c in range(NCHUNK):
            b, r = chunk(c)
            dma = pltpu.make_async_copy(
                o_ref.at[b, :, pl.ds(my_x * S_out + my_y * Q + r, R)],
                o_vm.at[NCHUNK + c],
                in_dma_sems.at[NCHUNK + c],
            )
            dma.start()
            in_dmas.append(dma)

        barrier = pltpu.get_barrier_semaphore()
        pl.semaphore_signal(
            barrier, inc=1, device_id=(peer_x, my_y),
            device_id_type=pl.DeviceIdType.MESH,
        )
        pl.semaphore_signal(
            barrier, inc=1, device_id=(my_x, peer_y),
            device_id_type=pl.DeviceIdType.MESH,
        )
        pl.semaphore_wait(barrier, 2)

        wo_bf[...] = wo_ref[...].astype(jnp.bfloat16)

        def dot_t(a_t):
            return lax.dot_general(
                a_t.astype(jnp.bfloat16), wo_bf[...],
                dimension_numbers=(((0,), (0,)), ((), ())),
                preferred_element_type=jnp.float32,
            )

        x_rdmas = []
        for c in range(NCHUNK):
            in_dmas[c].wait()
            x_send[c, :, :] = dot_t(o_vm[c]).astype(jnp.bfloat16)
            rdma = pltpu.make_async_remote_copy(
                src_ref=x_send.at[c],
                dst_ref=x_recv.at[c],
                send_sem=x_send_sems.at[c],
                recv_sem=x_recv_sems.at[c],
                device_id=(peer_x, my_y),
                device_id_type=pl.DeviceIdType.MESH,
            )
            rdma.start()
            x_rdmas.append(rdma)

        for c in range(NCHUNK):
            b, r = chunk(c)
            in_dmas[NCHUNK + c].wait()
            acc[b, pl.ds(r, R), :] = dot_t(o_vm[NCHUNK + c])

        y_rdmas = []
        out_dmas = []
        for c in range(NCHUNK):
            b, r = chunk(c)
            x_rdmas[c].wait_recv()
            sl = pl.ds(r, R)
            s = acc[b, sl, :] + x_recv[c, :, :].astype(jnp.float32)
            acc[b, sl, :] = s
            y_send[c, :, :] = s.astype(jnp.bfloat16)
            rdma = pltpu.make_async_remote_copy(
                src_ref=y_send.at[c],
                dst_ref=y_recv.at[c],
                send_sem=y_send_sems.at[c],
                recv_sem=y_recv_sems.at[c],
                device_id=(my_x, peer_y),
                device_id_type=pl.DeviceIdType.MESH,
            )
            rdma.start()
            y_rdmas.append(rdma)
            dma = pltpu.make_async_copy(
                acc.at[b, sl, :],
                out_ref.at[b, pl.ds(my_y * Q + r, R), :],
                out_dma_sems.at[0, c],
            )
            dma.start()
            out_dmas.append(dma)

        for c in range(NCHUNK):
            b, r = chunk(c)
            y_rdmas[c].wait_recv()
            qb[c, :, :] = y_recv[c, :, :].astype(jnp.float32)
            dma = pltpu.make_async_copy(
                qb.at[c],
                out_ref.at[b, pl.ds(peer_y * Q + r, R), :],
                out_dma_sems.at[1, c],
            )
            dma.start()
            out_dmas.append(dma)

        for dma in out_dmas:
            dma.wait()
        for c in range(NCHUNK):
            x_rdmas[c].wait_send()
            y_rdmas[c].wait_send()

    return pl.pallas_call(
        body,
        out_shape=jax.ShapeDtypeStruct((B, S_out, N), jnp.float32),
        in_specs=[
            pl.BlockSpec(memory_space=pltpu.MemorySpace.HBM),
            pl.BlockSpec(memory_space=pltpu.VMEM),
        ],
        out_specs=pl.BlockSpec(memory_space=pltpu.MemorySpace.HBM),
        scratch_shapes=[
            pltpu.VMEM((K, N), jnp.bfloat16),
            pltpu.VMEM((2 * NCHUNK, K, R), jnp.float32),
            pltpu.VMEM((NCHUNK, R, N), jnp.bfloat16),
            pltpu.VMEM((NCHUNK, R, N), jnp.bfloat16),
            pltpu.VMEM((NCHUNK, R, N), jnp.bfloat16),
            pltpu.VMEM((NCHUNK, R, N), jnp.bfloat16),
            pltpu.VMEM((B, Q, N), jnp.float32),
            pltpu.VMEM((NCHUNK, R, N), jnp.float32),
            pltpu.SemaphoreType.DMA((2 * NCHUNK,)),
            pltpu.SemaphoreType.DMA((NCHUNK,)),
            pltpu.SemaphoreType.DMA((NCHUNK,)),
            pltpu.SemaphoreType.DMA((NCHUNK,)),
            pltpu.SemaphoreType.DMA((NCHUNK,)),
            pltpu.SemaphoreType.DMA((2, NCHUNK)),
        ],
        compiler_params=pltpu.CompilerParams(
            collective_id=0, vmem_limit_bytes=64 * 1024 * 1024,
        ),
    )(OT, Wo)


# device time: 46362 ns/iter; 1.0223x vs baseline; 1.0223x over previous
import jax
import jax.numpy as jnp
from jax import lax
from jax.experimental import pallas as pl
from jax.experimental.pallas import tpu as pltpu

NCHUNK = 4


def kernel(O, Wo):
    B, S, H, D = O.shape
    K = H * D
    N = Wo.shape[1]
    S_out = S // 2
    Q = S_out // 2
    SPLIT = NCHUNK // B
    R = Q // SPLIT

    OT = O.transpose(0, 2, 3, 1).reshape(B, K, S)

    def body(o_ref, wo_ref, out_ref, wo_bf, o_vm, x_send, x_recv, acc,
             in_dma_sems, x_send_sems, x_recv_sems, y_send_sems, y_recv_sems):
        my_x = lax.axis_index("x")
        my_y = lax.axis_index("y")
        peer_x = 1 - my_x
        peer_y = 1 - my_y

        def chunk(c):
            return c // SPLIT, (c % SPLIT) * R

        in_dmas = []
        for c in range(NCHUNK):
            b, r = chunk(c)
            dma = pltpu.make_async_copy(
                o_ref.at[b, :, pl.ds(peer_x * S_out + my_y * Q + r, R)],
                o_vm.at[c],
                in_dma_sems.at[c],
            )
            dma.start()
            in_dmas.append(dma)
        for c in range(NCHUNK):
            b, r = chunk(c)
            dma = pltpu.make_async_copy(
                o_ref.at[b, :, pl.ds(my_x * S_out + my_y * Q + r, R)],
                o_vm.at[NCHUNK + c],
                in_dma_sems.at[NCHUNK + c],
            )
            dma.start()
            in_dmas.append(dma)

        barrier = pltpu.get_barrier_semaphore()
        pl.semaphore_signal(
            barrier, inc=1, device_id=(peer_x, my_y),
            device_id_type=pl.DeviceIdType.MESH,
        )
        pl.semaphore_signal(
            barrier, inc=1, device_id=(my_x, peer_y),
            device_id_type=pl.DeviceIdType.MESH,
        )
        pl.semaphore_wait(barrier, 2)

        wo_bf[...] = wo_ref[...].astype(jnp.bfloat16)

        def dot_t(a_t):
            return lax.dot_general(
                a_t.astype(jnp.bfloat16), wo_bf[...],
                dimension_numbers=(((0,), (0,)), ((), ())),
                preferred_element_type=jnp.float32,
            )

        x_rdmas = []
        for c in range(NCHUNK):
            in_dmas[c].wait()
            x_send[c, :, :] = dot_t(o_vm[c]).astype(jnp.bfloat16)
            rdma = pltpu.make_async_remote_copy(
                src_ref=x_send.at[c],
                dst_ref=x_recv.at[c],
                send_sem=x_send_sems.at[c],
                recv_sem=x_recv_sems.at[c],
                device_id=(peer_x, my_y),
                device_id_type=pl.DeviceIdType.MESH,
            )
            rdma.start()
            x_rdmas.append(rdma)

        for c in range(NCHUNK):
            b, r = chunk(c)
            in_dmas[NCHUNK + c].wait()
            acc[b, pl.ds(r, R), :] = dot_t(o_vm[NCHUNK + c])

        y_rdmas = []
        for c in range(NCHUNK):
            b, r = chunk(c)
            x_rdmas[c].wait_recv()
            out_sl = pl.ds(my_y * Q + r, R)
            out_ref[b, out_sl, :] = (
                acc[b, pl.ds(r, R), :] + x_recv[c, :, :].astype(jnp.float32)
            ).astype(jnp.bfloat16)
            rdma = pltpu.make_async_remote_copy(
                src_ref=out_ref.at[b, out_sl, :],
                dst_ref=out_ref.at[b, out_sl, :],
                send_sem=y_send_sems.at[c],
                recv_sem=y_recv_sems.at[c],
                device_id=(my_x, peer_y),
                device_id_type=pl.DeviceIdType.MESH,
            )
            rdma.start()
            y_rdmas.append(rdma)

        for c in range(NCHUNK):
            y_rdmas[c].wait_recv()

        for c in range(NCHUNK):
            x_rdmas[c].wait_send()
            y_rdmas[c].wait_send()

    return pl.pallas_call(
        body,
        out_shape=jax.ShapeDtypeStruct((B, S_out, N), jnp.bfloat16),
        in_specs=[
            pl.BlockSpec(memory_space=pltpu.MemorySpace.HBM),
            pl.BlockSpec(memory_space=pltpu.VMEM),
        ],
        out_specs=pl.BlockSpec(memory_space=pltpu.VMEM),
        scratch_shapes=[
            pltpu.VMEM((K, N), jnp.bfloat16),
            pltpu.VMEM((2 * NCHUNK, K, R), jnp.float32),
            pltpu.VMEM((NCHUNK, R, N), jnp.bfloat16),
            pltpu.VMEM((NCHUNK, R, N), jnp.bfloat16),
            pltpu.VMEM((B, Q, N), jnp.float32),
            pltpu.SemaphoreType.DMA((2 * NCHUNK,)),
            pltpu.SemaphoreType.DMA((NCHUNK,)),
            pltpu.SemaphoreType.DMA((NCHUNK,)),
            pltpu.SemaphoreType.DMA((NCHUNK,)),
            pltpu.SemaphoreType.DMA((NCHUNK,)),
        ],
        compiler_params=pltpu.CompilerParams(
            collective_id=0, vmem_limit_bytes=64 * 1024 * 1024,
        ),
    )(OT, Wo)
